# baseline (speedup 1.0000x reference)
import functools

import jax
import jax.numpy as jnp
from jax import lax
from jax.experimental import pallas as pl
from jax.experimental.pallas import tpu as pltpu
from jax.experimental.pallas import tpu_sc as plsc

NC, NS, L = 2, 16, 16
NW = NC * NS
IN_DIM = 128
PK = IN_DIM // 2
CB = 128
GROUPS = CB // L


@functools.lru_cache(maxsize=None)
def _build(n_rel, e_pad):
    sub_per_rel = NW // n_rel
    e_per_sub = e_pad // sub_per_rel
    n_chunks = e_per_sub // CB
    assert n_chunks % 4 == 0
    mesh = plsc.VectorSubcoreMesh(core_axis_name="c", subcore_axis_name="s")

    @functools.partial(
        pl.kernel,
        out_type=jax.ShapeDtypeStruct((n_rel * e_pad,), jnp.float32),
        mesh=mesh,
        compiler_params=pltpu.CompilerParams(
            needs_layout_passes=False, use_tc_tiling_on_sc=False),
        scratch_types=[
            pltpu.VMEM((e_per_sub // CB + 3, CB), jnp.int32),
            pltpu.VMEM((e_per_sub // CB + 3, CB), jnp.int32),
            pltpu.VMEM((CB, PK), jnp.float32),
            pltpu.VMEM((CB, PK), jnp.float32),
            pltpu.VMEM((CB, PK), jnp.float32),
            pltpu.VMEM((CB, PK), jnp.float32),
            pltpu.VMEM((CB, PK), jnp.float32),
            pltpu.VMEM((CB, PK), jnp.float32),
            pltpu.VMEM((CB, PK), jnp.float32),
            pltpu.VMEM((CB, PK), jnp.float32),
            pltpu.VMEM((PK,), jnp.float32),
            pltpu.VMEM((e_per_sub,), jnp.float32),
            pltpu.SemaphoreType.DMA,
            pltpu.SemaphoreType.DMA,
            pltpu.SemaphoreType.DMA,
            pltpu.SemaphoreType.DMA,
        ],
    )
    def decode(z_hbm, src_hbm, dst_hbm, w_hbm, out_hbm,
               si, di, sr0, dr0, sr1, dr1, sr2, dr2, sr3, dr3,
               w_v, o_all, sem0, sem1, sem2, sem3):
        wid = lax.axis_index("s") * NC + lax.axis_index("c")
        r = wid // sub_per_rel
        base = wid * e_per_sub

        pltpu.sync_copy(w_hbm.at[r], w_v)
        row0 = wid * n_chunks
        pltpu.sync_copy(src_hbm.at[pl.ds(row0, n_chunks + 3)], si)
        pltpu.sync_copy(dst_hbm.at[pl.ds(row0, n_chunks + 3)], di)

        bufs = ((sr0, dr0, sem0), (sr1, dr1, sem1),
                (sr2, dr2, sem2), (sr3, dr3, sem3))
        nbuf = len(bufs)
        lane = lax.iota(jnp.int32, L)

        def start_gather(c, p):
            sr, dr, sem = bufs[p]
            pltpu.async_copy(z_hbm.at[si.at[c]], sr, sem)
            pltpu.async_copy(z_hbm.at[di.at[c]], dr, sem)

        def wait_gather(c, p):
            sr, dr, sem = bufs[p]
            pltpu.make_async_copy(z_hbm.at[si.at[c]], sr, sem).wait()
            pltpu.make_async_copy(z_hbm.at[di.at[c]], dr, sem).wait()

        n_k = PK // L
        wab = []
        for k in range(n_k):
            wbf = plsc.bitcast(w_v[pl.ds(k * L, L)], jnp.bfloat16)
            wab.append(plsc.unpack(wbf, format=plsc.PackFormat.INTERLEAVED))

        def compute(c, p):
            sr, dr, _ = bufs[p]

            def g_body(g, _):
                ebase = g * L

                def q_body(q, ovec):
                    for j in range(4):
                        e = q * 4 + j
                        acc = None
                        for k in range(n_k):
                            sb = plsc.bitcast(sr[ebase + e, pl.ds(k * L, L)],
                                              jnp.bfloat16)
                            db = plsc.bitcast(dr[ebase + e, pl.ds(k * L, L)],
                                              jnp.bfloat16)
                            pa, pb = plsc.unpack(sb * db,
                                                 format=plsc.PackFormat.INTERLEAVED)
                            wa, wb = wab[k]
                            t = pa * wa + pb * wb
                            acc = t if acc is None else acc + t
                        ovec = jnp.where(lane == e, jnp.sum(acc), ovec)
                    return ovec

                ovec = lax.fori_loop(0, 4, q_body, jnp.zeros((L,), jnp.float32))
                o_all[pl.ds(c * CB + g * L, L)] = 1.0 / (1.0 + jnp.exp(-ovec))
                return 0

            lax.fori_loop(0, GROUPS, g_body, 0)

        for c in range(nbuf - 1):
            start_gather(c, c)

        def round_body(i, _):
            c0 = i * nbuf
            for b in range(nbuf):
                c = c0 + b
                start_gather(c + nbuf - 1, (b + nbuf - 1) % nbuf)
                wait_gather(c, b)
                compute(c, b)
            return 0

        lax.fori_loop(0, n_chunks // nbuf, round_body, 0)
        for k in range(nbuf - 1):
            wait_gather(n_chunks + k, (n_chunks + k) % nbuf)
        pltpu.sync_copy(o_all, out_hbm.at[pl.ds(base, e_per_sub)])

    return decode


def kernel(z, edge_index, weight):
    n_rel, _, e = edge_index.shape
    sub_per_rel = NW // n_rel
    quantum = sub_per_rel * CB * 4
    e_pad = ((e + quantum - 1) // quantum) * quantum
    idx = edge_index.astype(jnp.int32)
    src = jnp.pad(idx[:, 0, :], ((0, 0), (0, e_pad - e))).reshape(-1)
    dst = jnp.pad(idx[:, 1, :], ((0, 0), (0, e_pad - e))).reshape(-1)
    src = jnp.pad(src, (0, 3 * CB)).reshape(-1, CB)
    dst = jnp.pad(dst, (0, 3 * CB)).reshape(-1, CB)
    z_pk = lax.bitcast_convert_type(
        z.astype(jnp.bfloat16).reshape(z.shape[0], PK, 2), jnp.float32)
    w_pk = lax.bitcast_convert_type(
        weight.astype(jnp.bfloat16).reshape(weight.shape[0], PK, 2), jnp.float32)
    out = _build(n_rel, e_pad)(z_pk, src, dst, w_pk)
    return out.reshape(n_rel, e_pad)[:, :e]


# speedup vs baseline: 1.5309x; 1.5309x over previous
import functools

import jax
import jax.numpy as jnp
from jax import lax
from jax.experimental import pallas as pl
from jax.experimental.pallas import tpu as pltpu
from jax.experimental.pallas import tpu_sc as plsc

NC, NS, L = 2, 16, 16
NW = NC * NS
IN_DIM = 128
PK = IN_DIM // 2
CB = 128
GROUPS = CB // L


@functools.lru_cache(maxsize=None)
def _build(n_rel, e_pad):
    sub_per_rel = NW // n_rel
    e_per_sub = e_pad // sub_per_rel
    n_chunks = e_per_sub // CB
    assert n_chunks % 4 == 0
    mesh = plsc.VectorSubcoreMesh(core_axis_name="c", subcore_axis_name="s")

    @functools.partial(
        pl.kernel,
        out_type=jax.ShapeDtypeStruct((n_rel * e_pad,), jnp.float32),
        mesh=mesh,
        compiler_params=pltpu.CompilerParams(
            needs_layout_passes=False, use_tc_tiling_on_sc=False),
        scratch_types=[
            pltpu.VMEM((e_per_sub + 3 * CB,), jnp.int32),
            pltpu.VMEM((e_per_sub + 3 * CB,), jnp.int32),
            pltpu.VMEM((CB, PK), jnp.float32),
            pltpu.VMEM((CB, PK), jnp.float32),
            pltpu.VMEM((CB, PK), jnp.float32),
            pltpu.VMEM((CB, PK), jnp.float32),
            pltpu.VMEM((CB, PK), jnp.float32),
            pltpu.VMEM((CB, PK), jnp.float32),
            pltpu.VMEM((CB, PK), jnp.float32),
            pltpu.VMEM((CB, PK), jnp.float32),
            pltpu.VMEM((PK,), jnp.float32),
            pltpu.VMEM((e_per_sub,), jnp.float32),
            pltpu.SemaphoreType.DMA,
            pltpu.SemaphoreType.DMA,
            pltpu.SemaphoreType.DMA,
            pltpu.SemaphoreType.DMA,
        ],
    )
    def decode(z_hbm, src_hbm, dst_hbm, w_hbm, out_hbm,
               si, di, sr0, dr0, sr1, dr1, sr2, dr2, sr3, dr3,
               w_v, o_all, sem0, sem1, sem2, sem3):
        wid = lax.axis_index("s") * NC + lax.axis_index("c")
        r = wid // sub_per_rel
        base = wid * e_per_sub

        pltpu.sync_copy(w_hbm.at[r], w_v)
        pltpu.sync_copy(src_hbm.at[pl.ds(base, e_per_sub + 3 * CB)], si)
        pltpu.sync_copy(dst_hbm.at[pl.ds(base, e_per_sub + 3 * CB)], di)

        bufs = ((sr0, dr0, sem0), (sr1, dr1, sem1),
                (sr2, dr2, sem2), (sr3, dr3, sem3))
        nbuf = len(bufs)
        lane = lax.iota(jnp.int32, L)

        def start_gather(c, p):
            sr, dr, sem = bufs[p]
            pltpu.async_copy(z_hbm.at[si.at[pl.ds(c * CB, CB)]], sr, sem)
            pltpu.async_copy(z_hbm.at[di.at[pl.ds(c * CB, CB)]], dr, sem)

        def wait_gather(c, p):
            sr, dr, sem = bufs[p]
            pltpu.make_async_copy(z_hbm.at[si.at[pl.ds(c * CB, CB)]], sr, sem).wait()
            pltpu.make_async_copy(z_hbm.at[di.at[pl.ds(c * CB, CB)]], dr, sem).wait()

        n_k = PK // L
        wab = []
        for k in range(n_k):
            wbf = plsc.bitcast(w_v[pl.ds(k * L, L)], jnp.bfloat16)
            wab.append(plsc.unpack(wbf, format=plsc.PackFormat.INTERLEAVED))

        def compute(c, p):
            sr, dr, _ = bufs[p]

            def g_body(g, _):
                ebase = g * L

                def q_body(q, ovec):
                    for j in range(4):
                        e = q * 4 + j
                        acc = None
                        for k in range(n_k):
                            sb = plsc.bitcast(sr[ebase + e, pl.ds(k * L, L)],
                                              jnp.bfloat16)
                            db = plsc.bitcast(dr[ebase + e, pl.ds(k * L, L)],
                                              jnp.bfloat16)
                            pa, pb = plsc.unpack(sb * db,
                                                 format=plsc.PackFormat.INTERLEAVED)
                            wa, wb = wab[k]
                            t = pa * wa + pb * wb
                            acc = t if acc is None else acc + t
                        ovec = jnp.where(lane == e, jnp.sum(acc), ovec)
                    return ovec

                ovec = lax.fori_loop(0, 4, q_body, jnp.zeros((L,), jnp.float32))
                o_all[pl.ds(c * CB + g * L, L)] = 1.0 / (1.0 + jnp.exp(-ovec))
                return 0

            lax.fori_loop(0, GROUPS, g_body, 0)

        for c in range(nbuf - 1):
            start_gather(c, c)

        def round_body(i, _):
            c0 = i * nbuf
            for b in range(nbuf):
                c = c0 + b
                start_gather(c + nbuf - 1, (b + nbuf - 1) % nbuf)
                wait_gather(c, b)
                compute(c, b)
            return 0

        lax.fori_loop(0, n_chunks // nbuf, round_body, 0)
        for k in range(nbuf - 1):
            wait_gather(n_chunks + k, (n_chunks + k) % nbuf)
        pltpu.sync_copy(o_all, out_hbm.at[pl.ds(base, e_per_sub)])

    return decode


def kernel(z, edge_index, weight):
    n_rel, _, e = edge_index.shape
    sub_per_rel = NW // n_rel
    quantum = sub_per_rel * CB * 4
    e_pad = ((e + quantum - 1) // quantum) * quantum
    idx = edge_index.astype(jnp.int32)
    fill = jnp.arange(n_rel * (e_pad - e), dtype=jnp.int32).reshape(
        n_rel, e_pad - e) % z.shape[0]
    src = jnp.concatenate([idx[:, 0, :], fill], axis=1).reshape(-1)
    dst = jnp.concatenate([idx[:, 1, :], fill], axis=1).reshape(-1)
    tail = jnp.arange(3 * CB, dtype=jnp.int32) % z.shape[0]
    src = jnp.concatenate([src, tail])
    dst = jnp.concatenate([dst, tail])
    z_pk = lax.bitcast_convert_type(
        z.astype(jnp.bfloat16).reshape(z.shape[0], PK, 2), jnp.float32)
    w_pk = lax.bitcast_convert_type(
        weight.astype(jnp.bfloat16).reshape(weight.shape[0], PK, 2), jnp.float32)
    out = _build(n_rel, e_pad)(z_pk, src, dst, w_pk)
    return out.reshape(n_rel, e_pad)[:, :e]


# speedup vs baseline: 2.6503x; 1.7312x over previous
import functools

import jax
import jax.numpy as jnp
from jax import lax
from jax.experimental import pallas as pl
from jax.experimental.pallas import tpu as pltpu
from jax.experimental.pallas import tpu_sc as plsc

NC, NS, L = 2, 16, 16
NW = NC * NS
IN_DIM = 128
PK = IN_DIM // 2
CB = 128
GROUPS = CB // L


@functools.lru_cache(maxsize=None)
def _build(n_rel, e_pad):
    sub_per_rel = NW // n_rel
    e_per_sub = e_pad // sub_per_rel
    n_chunks = e_per_sub // CB
    assert n_chunks % 4 == 0
    mesh = plsc.VectorSubcoreMesh(core_axis_name="c", subcore_axis_name="s")

    @functools.partial(
        pl.kernel,
        out_type=jax.ShapeDtypeStruct((n_rel * e_pad,), jnp.float32),
        mesh=mesh,
        compiler_params=pltpu.CompilerParams(
            needs_layout_passes=False, use_tc_tiling_on_sc=False),
        scratch_types=[
            pltpu.VMEM((e_per_sub + 3 * CB,), jnp.int32),
            pltpu.VMEM((e_per_sub + 3 * CB,), jnp.int32),
            pltpu.VMEM((CB, PK), jnp.float32),
            pltpu.VMEM((CB, PK), jnp.float32),
            pltpu.VMEM((CB, PK), jnp.float32),
            pltpu.VMEM((CB, PK), jnp.float32),
            pltpu.VMEM((CB, PK), jnp.float32),
            pltpu.VMEM((CB, PK), jnp.float32),
            pltpu.VMEM((CB, PK), jnp.float32),
            pltpu.VMEM((CB, PK), jnp.float32),
            pltpu.VMEM((IN_DIM,), jnp.float32),
            pltpu.VMEM((e_per_sub,), jnp.float32),
            pltpu.SemaphoreType.DMA,
            pltpu.SemaphoreType.DMA,
            pltpu.SemaphoreType.DMA,
            pltpu.SemaphoreType.DMA,
        ],
    )
    def decode(z_hbm, src_hbm, dst_hbm, w_hbm, out_hbm,
               si, di, sr0, dr0, sr1, dr1, sr2, dr2, sr3, dr3,
               w_v, o_all, sem0, sem1, sem2, sem3):
        wid = lax.axis_index("s") * NC + lax.axis_index("c")
        r = wid // sub_per_rel
        base = wid * e_per_sub

        pltpu.sync_copy(w_hbm.at[r], w_v)
        pltpu.sync_copy(src_hbm.at[pl.ds(base, e_per_sub + 3 * CB)], si)
        pltpu.sync_copy(dst_hbm.at[pl.ds(base, e_per_sub + 3 * CB)], di)

        bufs = ((sr0, dr0, sem0), (sr1, dr1, sem1),
                (sr2, dr2, sem2), (sr3, dr3, sem3))
        nbuf = len(bufs)
        lane = lax.iota(jnp.int32, L)

        def start_gather(c, p):
            sr, dr, sem = bufs[p]
            pltpu.async_copy(z_hbm.at[si.at[pl.ds(c * CB, CB)]], sr, sem)
            pltpu.async_copy(z_hbm.at[di.at[pl.ds(c * CB, CB)]], dr, sem)

        def wait_gather(c, p):
            sr, dr, sem = bufs[p]
            pltpu.make_async_copy(z_hbm.at[si.at[pl.ds(c * CB, CB)]], sr, sem).wait()
            pltpu.make_async_copy(z_hbm.at[di.at[pl.ds(c * CB, CB)]], dr, sem).wait()

        n_k = PK // L
        wab = [(w_v[pl.ds(k * L, L)], w_v[pl.ds(PK + k * L, L)])
               for k in range(n_k)]

        def compute(c, p):
            sr, dr, _ = bufs[p]

            def g_body(g, _):
                ebase = g * L

                def q_body(q, ovec):
                    for j in range(4):
                        e = q * 4 + j
                        acc = None
                        for k in range(n_k):
                            sb = plsc.bitcast(sr[ebase + e, pl.ds(k * L, L)],
                                              jnp.bfloat16)
                            db = plsc.bitcast(dr[ebase + e, pl.ds(k * L, L)],
                                              jnp.bfloat16)
                            pa, pb = plsc.unpack(sb * db,
                                                 format=plsc.PackFormat.INTERLEAVED)
                            wa, wb = wab[k]
                            t = pa * wa + pb * wb
                            acc = t if acc is None else acc + t
                        ovec = jnp.where(lane == e, jnp.sum(acc), ovec)
                    return ovec

                ovec = lax.fori_loop(0, 4, q_body, jnp.zeros((L,), jnp.float32))
                o_all[pl.ds(c * CB + g * L, L)] = 1.0 / (1.0 + jnp.exp(-ovec))
                return 0

            lax.fori_loop(0, GROUPS, g_body, 0)

        for c in range(nbuf - 1):
            start_gather(c, c)

        def round_body(i, _):
            c0 = i * nbuf
            for b in range(nbuf):
                c = c0 + b
                start_gather(c + nbuf - 1, (b + nbuf - 1) % nbuf)
                wait_gather(c, b)
                compute(c, b)
            return 0

        lax.fori_loop(0, n_chunks // nbuf, round_body, 0)
        for k in range(nbuf - 1):
            wait_gather(n_chunks + k, (n_chunks + k) % nbuf)
        pltpu.sync_copy(o_all, out_hbm.at[pl.ds(base, e_per_sub)])

    return decode


def kernel(z, edge_index, weight):
    n_rel, _, e = edge_index.shape
    sub_per_rel = NW // n_rel
    quantum = sub_per_rel * CB * 4
    e_pad = ((e + quantum - 1) // quantum) * quantum
    idx = edge_index.astype(jnp.int32)
    fill = jnp.arange(n_rel * (e_pad - e), dtype=jnp.int32).reshape(
        n_rel, e_pad - e) % z.shape[0]
    src = jnp.concatenate([idx[:, 0, :], fill], axis=1).reshape(-1)
    dst = jnp.concatenate([idx[:, 1, :], fill], axis=1).reshape(-1)
    tail = jnp.arange(3 * CB, dtype=jnp.int32) % z.shape[0]
    src = jnp.concatenate([src, tail])
    dst = jnp.concatenate([dst, tail])
    u = lax.bitcast_convert_type(z, jnp.uint32)
    rnd = lambda x: (x + 0x7FFF + ((x >> 16) & 1)) >> 16
    pk = rnd(u[:, :PK]) | (rnd(u[:, PK:]) << 16)
    z_pk = lax.bitcast_convert_type(pk, jnp.float32)
    out = _build(n_rel, e_pad)(z_pk, src, dst, weight.astype(jnp.float32))
    return out.reshape(n_rel, e_pad)[:, :e]


# speedup vs baseline: 2.6532x; 1.0011x over previous
import functools

import jax
import jax.numpy as jnp
from jax import lax
from jax.experimental import pallas as pl
from jax.experimental.pallas import tpu as pltpu
from jax.experimental.pallas import tpu_sc as plsc

NC, NS, L = 2, 16, 16
NW = NC * NS
IN_DIM = 128
PK = IN_DIM // 2
CB = 128
GROUPS = CB // L


@functools.lru_cache(maxsize=None)
def _build(n_rel, e_pad, E_REAL):
    sub_per_rel = NW // n_rel
    e_per_sub = e_pad // sub_per_rel
    last_len = E_REAL - (sub_per_rel - 1) * e_per_sub
    assert 0 < last_len <= e_per_sub and last_len % 8 == 0
    n_chunks = e_per_sub // CB
    assert n_chunks % 4 == 0
    mesh = plsc.VectorSubcoreMesh(core_axis_name="c", subcore_axis_name="s")

    @functools.partial(
        pl.kernel,
        out_type=jax.ShapeDtypeStruct((n_rel * E_REAL,), jnp.float32),
        mesh=mesh,
        compiler_params=pltpu.CompilerParams(
            needs_layout_passes=False, use_tc_tiling_on_sc=False),
        scratch_types=[
            pltpu.VMEM((e_per_sub + 3 * CB,), jnp.int32),
            pltpu.VMEM((e_per_sub + 3 * CB,), jnp.int32),
            pltpu.VMEM((CB, PK), jnp.float32),
            pltpu.VMEM((CB, PK), jnp.float32),
            pltpu.VMEM((CB, PK), jnp.float32),
            pltpu.VMEM((CB, PK), jnp.float32),
            pltpu.VMEM((CB, PK), jnp.float32),
            pltpu.VMEM((CB, PK), jnp.float32),
            pltpu.VMEM((CB, PK), jnp.float32),
            pltpu.VMEM((CB, PK), jnp.float32),
            pltpu.VMEM((IN_DIM,), jnp.float32),
            pltpu.VMEM((e_per_sub,), jnp.float32),
            pltpu.SemaphoreType.DMA,
            pltpu.SemaphoreType.DMA,
            pltpu.SemaphoreType.DMA,
            pltpu.SemaphoreType.DMA,
        ],
    )
    def decode(z_hbm, src_hbm, dst_hbm, w_hbm, out_hbm,
               si, di, sr0, dr0, sr1, dr1, sr2, dr2, sr3, dr3,
               w_v, o_all, sem0, sem1, sem2, sem3):
        wid = lax.axis_index("s") * NC + lax.axis_index("c")
        r = wid // sub_per_rel
        base = wid * e_per_sub

        pltpu.sync_copy(w_hbm.at[r], w_v)
        pltpu.sync_copy(src_hbm.at[pl.ds(base, e_per_sub + 3 * CB)], si)
        pltpu.sync_copy(dst_hbm.at[pl.ds(base, e_per_sub + 3 * CB)], di)

        bufs = ((sr0, dr0, sem0), (sr1, dr1, sem1),
                (sr2, dr2, sem2), (sr3, dr3, sem3))
        nbuf = len(bufs)
        lane = lax.iota(jnp.int32, L)

        def start_gather(c, p):
            sr, dr, sem = bufs[p]
            pltpu.async_copy(z_hbm.at[si.at[pl.ds(c * CB, CB)]], sr, sem)
            pltpu.async_copy(z_hbm.at[di.at[pl.ds(c * CB, CB)]], dr, sem)

        def wait_gather(c, p):
            sr, dr, sem = bufs[p]
            pltpu.make_async_copy(z_hbm.at[si.at[pl.ds(c * CB, CB)]], sr, sem).wait()
            pltpu.make_async_copy(z_hbm.at[di.at[pl.ds(c * CB, CB)]], dr, sem).wait()

        n_k = PK // L
        wab = [(w_v[pl.ds(k * L, L)], w_v[pl.ds(PK + k * L, L)])
               for k in range(n_k)]

        def compute(c, p):
            sr, dr, _ = bufs[p]

            def g_body(g, _):
                ebase = g * L

                def q_body(q, ovec):
                    for j in range(4):
                        e = q * 4 + j
                        acc = None
                        for k in range(n_k):
                            sb = plsc.bitcast(sr[ebase + e, pl.ds(k * L, L)],
                                              jnp.bfloat16)
                            db = plsc.bitcast(dr[ebase + e, pl.ds(k * L, L)],
                                              jnp.bfloat16)
                            pa, pb = plsc.unpack(sb * db,
                                                 format=plsc.PackFormat.INTERLEAVED)
                            wa, wb = wab[k]
                            t = pa * wa + pb * wb
                            acc = t if acc is None else acc + t
                        ovec = jnp.where(lane == e, jnp.sum(acc), ovec)
                    return ovec

                ovec = lax.fori_loop(0, 4, q_body, jnp.zeros((L,), jnp.float32))
                o_all[pl.ds(c * CB + g * L, L)] = 1.0 / (1.0 + jnp.exp(-ovec))
                return 0

            lax.fori_loop(0, GROUPS, g_body, 0)

        for c in range(nbuf - 1):
            start_gather(c, c)

        def round_body(i, _):
            c0 = i * nbuf
            for b in range(nbuf):
                c = c0 + b
                start_gather(c + nbuf - 1, (b + nbuf - 1) % nbuf)
                wait_gather(c, b)
                compute(c, b)
            return 0

        lax.fori_loop(0, n_chunks // nbuf, round_body, 0)
        for k in range(nbuf - 1):
            wait_gather(n_chunks + k, (n_chunks + k) % nbuf)
        sub = wid % sub_per_rel
        obase = r * E_REAL + sub * e_per_sub
        if last_len == e_per_sub:
            pltpu.sync_copy(o_all, out_hbm.at[pl.ds(obase, e_per_sub)])
        else:
            @pl.when(sub < sub_per_rel - 1)
            def _():
                pltpu.sync_copy(o_all, out_hbm.at[pl.ds(obase, e_per_sub)])

            @pl.when(sub == sub_per_rel - 1)
            def _():
                pltpu.sync_copy(o_all.at[pl.ds(0, last_len)],
                                out_hbm.at[pl.ds(obase, last_len)])

    return decode


def kernel(z, edge_index, weight):
    n_rel, _, e = edge_index.shape
    sub_per_rel = NW // n_rel
    quantum = sub_per_rel * CB * 4
    e_pad = ((e + quantum - 1) // quantum) * quantum
    idx = edge_index.astype(jnp.int32)
    fill = jnp.arange(n_rel * (e_pad - e), dtype=jnp.int32).reshape(
        n_rel, e_pad - e) % z.shape[0]
    src = jnp.concatenate([idx[:, 0, :], fill], axis=1).reshape(-1)
    dst = jnp.concatenate([idx[:, 1, :], fill], axis=1).reshape(-1)
    tail = jnp.arange(3 * CB, dtype=jnp.int32) % z.shape[0]
    src = jnp.concatenate([src, tail])
    dst = jnp.concatenate([dst, tail])
    u = lax.bitcast_convert_type(z, jnp.uint32)
    rnd = lambda x: (x + 0x7FFF + ((x >> 16) & 1)) >> 16
    pk = rnd(u[:, :PK]) | (rnd(u[:, PK:]) << 16)
    z_pk = lax.bitcast_convert_type(pk, jnp.float32)
    out = _build(n_rel, e_pad, e)(z_pk, src, dst, weight.astype(jnp.float32))
    return out.reshape(n_rel, e)


# speedup vs baseline: 3.0162x; 1.1369x over previous
import functools

import jax
import jax.numpy as jnp
from jax import lax
from jax.experimental import pallas as pl
from jax.experimental.pallas import tpu as pltpu
from jax.experimental.pallas import tpu_sc as plsc

NC, NS, L = 2, 16, 16
NW = NC * NS
IN_DIM = 128
PK = IN_DIM // 2
CB = 128
GROUPS = CB // L


@functools.lru_cache(maxsize=None)
def _pack_build(n_rows):
    RC = 128
    n_chunks = (n_rows + RC - 1) // RC
    n_pad = n_chunks * RC
    mesh = plsc.VectorSubcoreMesh(core_axis_name="c", subcore_axis_name="s")

    @functools.partial(
        pl.kernel,
        out_type=jax.ShapeDtypeStruct((n_pad, PK), jnp.float32),
        mesh=mesh,
        compiler_params=pltpu.CompilerParams(
            needs_layout_passes=False, use_tc_tiling_on_sc=False),
        scratch_types=[
            pltpu.VMEM((RC, IN_DIM), jnp.float32),
            pltpu.VMEM((RC, IN_DIM), jnp.float32),
            pltpu.VMEM((RC, PK), jnp.float32),
            pltpu.SemaphoreType.DMA,
            pltpu.SemaphoreType.DMA,
        ],
    )
    def pack_k(z_hbm, out_hbm, zin0, zin1, zout, sem0, sem1):
        wid = lax.axis_index("s") * NC + lax.axis_index("c")
        ins = ((zin0, sem0), (zin1, sem1))
        n_mine = (n_chunks - wid + NW - 1) // NW

        def rstart(i):
            return jnp.minimum((wid + i * NW) * RC, n_rows - RC)

        def start_in(i, p):
            zin, sem = ins[p]
            pltpu.async_copy(z_hbm.at[pl.ds(rstart(i), RC)], zin, sem)

        def wait_in(i, p):
            zin, sem = ins[p]
            pltpu.make_async_copy(z_hbm.at[pl.ds(rstart(i), RC)], zin, sem).wait()

        @pl.when(n_mine > 0)
        def _():
            start_in(0, 0)

        n_pairs = ((n_chunks + NW - 1) // NW + 1) // 2

        def body(ip, _):
            for b in range(2):
                idx = ip * 2 + b

                @pl.when(idx < n_mine)
                def _():
                    @pl.when(idx + 1 < n_mine)
                    def _():
                        start_in(idx + 1, 1 - b)

                    wait_in(idx, b)
                    zin = ins[b][0]

                    def row_body(rr, _):
                        for k in range(PK // L):
                            va = zin[rr, pl.ds(k * L, L)]
                            vb = zin[rr, pl.ds(PK + k * L, L)]
                            pk16 = plsc.bitcast(
                                plsc.pack(va, vb,
                                          format=plsc.PackFormat.INTERLEAVED),
                                jnp.float32)
                            zout[rr, pl.ds(k * L, L)] = pk16
                        return 0

                    lax.fori_loop(0, RC, row_body, 0)
                    pltpu.sync_copy(zout, out_hbm.at[pl.ds(rstart(idx), RC)])
            return 0

        lax.fori_loop(0, n_pairs, body, 0)

    return pack_k


@functools.lru_cache(maxsize=None)
def _build(n_rel, e_pad, E_REAL):
    sub_per_rel = NW // n_rel
    e_per_sub = e_pad // sub_per_rel
    last_len = E_REAL - (sub_per_rel - 1) * e_per_sub
    assert 0 < last_len <= e_per_sub and last_len % 8 == 0
    n_chunks = e_per_sub // CB
    assert n_chunks % 4 == 0
    mesh = plsc.VectorSubcoreMesh(core_axis_name="c", subcore_axis_name="s")

    @functools.partial(
        pl.kernel,
        out_type=jax.ShapeDtypeStruct((n_rel * E_REAL,), jnp.float32),
        mesh=mesh,
        compiler_params=pltpu.CompilerParams(
            needs_layout_passes=False, use_tc_tiling_on_sc=False),
        scratch_types=[
            pltpu.VMEM((e_per_sub + 3 * CB,), jnp.int32),
            pltpu.VMEM((e_per_sub + 3 * CB,), jnp.int32),
            pltpu.VMEM((CB, PK), jnp.float32),
            pltpu.VMEM((CB, PK), jnp.float32),
            pltpu.VMEM((CB, PK), jnp.float32),
            pltpu.VMEM((CB, PK), jnp.float32),
            pltpu.VMEM((CB, PK), jnp.float32),
            pltpu.VMEM((CB, PK), jnp.float32),
            pltpu.VMEM((CB, PK), jnp.float32),
            pltpu.VMEM((CB, PK), jnp.float32),
            pltpu.VMEM((IN_DIM,), jnp.float32),
            pltpu.VMEM((e_per_sub,), jnp.float32),
            pltpu.SemaphoreType.DMA,
            pltpu.SemaphoreType.DMA,
            pltpu.SemaphoreType.DMA,
            pltpu.SemaphoreType.DMA,
        ],
    )
    def decode(z_hbm, src_hbm, dst_hbm, w_hbm, out_hbm,
               si, di, sr0, dr0, sr1, dr1, sr2, dr2, sr3, dr3,
               w_v, o_all, sem0, sem1, sem2, sem3):
        wid = lax.axis_index("s") * NC + lax.axis_index("c")
        r = wid // sub_per_rel
        base = wid * e_per_sub

        pltpu.sync_copy(w_hbm.at[r], w_v)
        pltpu.sync_copy(src_hbm.at[pl.ds(base, e_per_sub + 3 * CB)], si)
        pltpu.sync_copy(dst_hbm.at[pl.ds(base, e_per_sub + 3 * CB)], di)

        bufs = ((sr0, dr0, sem0), (sr1, dr1, sem1),
                (sr2, dr2, sem2), (sr3, dr3, sem3))
        nbuf = len(bufs)
        lane = lax.iota(jnp.int32, L)

        def start_gather(c, p):
            sr, dr, sem = bufs[p]
            pltpu.async_copy(z_hbm.at[si.at[pl.ds(c * CB, CB)]], sr, sem)
            pltpu.async_copy(z_hbm.at[di.at[pl.ds(c * CB, CB)]], dr, sem)

        def wait_gather(c, p):
            sr, dr, sem = bufs[p]
            pltpu.make_async_copy(z_hbm.at[si.at[pl.ds(c * CB, CB)]], sr, sem).wait()
            pltpu.make_async_copy(z_hbm.at[di.at[pl.ds(c * CB, CB)]], dr, sem).wait()

        n_k = PK // L
        wab = [(w_v[pl.ds(k * L, L)], w_v[pl.ds(PK + k * L, L)])
               for k in range(n_k)]

        def compute(c, p):
            sr, dr, _ = bufs[p]

            def g_body(g, _):
                ebase = g * L

                def q_body(q, ovec):
                    for j in range(4):
                        e = q * 4 + j
                        acc = None
                        for k in range(n_k):
                            sb = plsc.bitcast(sr[ebase + e, pl.ds(k * L, L)],
                                              jnp.bfloat16)
                            db = plsc.bitcast(dr[ebase + e, pl.ds(k * L, L)],
                                              jnp.bfloat16)
                            pa, pb = plsc.unpack(sb * db,
                                                 format=plsc.PackFormat.INTERLEAVED)
                            wa, wb = wab[k]
                            t = pa * wa + pb * wb
                            acc = t if acc is None else acc + t
                        ovec = jnp.where(lane == e, jnp.sum(acc), ovec)
                    return ovec

                ovec = lax.fori_loop(0, 4, q_body, jnp.zeros((L,), jnp.float32))
                o_all[pl.ds(c * CB + g * L, L)] = 1.0 / (1.0 + jnp.exp(-ovec))
                return 0

            lax.fori_loop(0, GROUPS, g_body, 0)

        for c in range(nbuf - 1):
            start_gather(c, c)

        def round_body(i, _):
            c0 = i * nbuf
            for b in range(nbuf):
                c = c0 + b
                start_gather(c + nbuf - 1, (b + nbuf - 1) % nbuf)
                wait_gather(c, b)
                compute(c, b)
            return 0

        lax.fori_loop(0, n_chunks // nbuf, round_body, 0)
        for k in range(nbuf - 1):
            wait_gather(n_chunks + k, (n_chunks + k) % nbuf)
        sub = wid % sub_per_rel
        obase = r * E_REAL + sub * e_per_sub
        if last_len == e_per_sub:
            pltpu.sync_copy(o_all, out_hbm.at[pl.ds(obase, e_per_sub)])
        else:
            @pl.when(sub < sub_per_rel - 1)
            def _():
                pltpu.sync_copy(o_all, out_hbm.at[pl.ds(obase, e_per_sub)])

            @pl.when(sub == sub_per_rel - 1)
            def _():
                pltpu.sync_copy(o_all.at[pl.ds(0, last_len)],
                                out_hbm.at[pl.ds(obase, last_len)])

    return decode


def kernel(z, edge_index, weight):
    n_rel, _, e = edge_index.shape
    sub_per_rel = NW // n_rel
    quantum = sub_per_rel * CB * 4
    e_pad = ((e + quantum - 1) // quantum) * quantum
    idx = edge_index.astype(jnp.int32)
    fill = jnp.arange(n_rel * (e_pad - e), dtype=jnp.int32).reshape(
        n_rel, e_pad - e) % z.shape[0]
    src = jnp.concatenate([idx[:, 0, :], fill], axis=1).reshape(-1)
    dst = jnp.concatenate([idx[:, 1, :], fill], axis=1).reshape(-1)
    tail = jnp.arange(3 * CB, dtype=jnp.int32) % z.shape[0]
    src = jnp.concatenate([src, tail])
    dst = jnp.concatenate([dst, tail])
    z_pk = _pack_build(z.shape[0])(z)
    out = _build(n_rel, e_pad, e)(z_pk, src, dst, weight.astype(jnp.float32))
    return out.reshape(n_rel, e)
